# baseline (speedup 1.0000x reference)
import functools

import jax
import jax.numpy as jnp
from jax import lax
from jax.experimental import pallas as pl
from jax.experimental.pallas import tpu as pltpu
from jax.experimental.pallas import tpu_sc as plsc

NUM_CORES = 2
NUM_SUBCORES = 16
LANES = 16
NW = NUM_CORES * NUM_SUBCORES
IDX_CHUNK = 128


def _fm_kernel(d_latent, b_per_w, ui_hbm, ii_hbm, eu_hbm, ei_hbm, b16_hbm,
               hrep_hbm, out_hbm, ui_v, ii_v, hi_v, lo_v, u_v, i_v, b_v, h_v,
               o_v, sem):
    wid = lax.axis_index("s") * NUM_CORES + lax.axis_index("c")
    n_chunks = b_per_w // IDX_CHUNK
    grp_per_chunk = IDX_CHUNK // LANES
    base_row = wid * n_chunks

    pltpu.sync_copy(ui_hbm.at[pl.ds(base_row, n_chunks)], ui_v)
    pltpu.sync_copy(ii_hbm.at[pl.ds(base_row, n_chunks)], ii_v)
    pltpu.sync_copy(hrep_hbm, h_v)

    for j in range(n_chunks):
        for t in range(grp_per_chunk):
            r = ii_v[j, pl.ds(t * LANES, LANES)]
            hi_v[j, pl.ds(t * LANES, LANES)] = lax.shift_right_logical(r, 4)
            lo_v[pl.ds((j * grp_per_chunk + t) * LANES, LANES)] = (
                jnp.bitwise_and(r, 15))

    copies = []
    for j in range(n_chunks):
        sl = pl.ds(j * IDX_CHUNK, IDX_CHUNK)
        copies.append(pltpu.async_copy(eu_hbm.at[ui_v.at[j]], u_v.at[sl], sem))
        copies.append(pltpu.async_copy(ei_hbm.at[ii_v.at[j]], i_v.at[sl], sem))
        copies.append(pltpu.async_copy(b16_hbm.at[hi_v.at[j]], b_v.at[sl], sem))
    for c in copies:
        c.wait()

    hs = [h_v[pl.ds(d * LANES, LANES)] for d in range(d_latent)]
    zeros = jnp.zeros((LANES,), jnp.int32)

    def group(g, carry):
        bidx = g * LANES + lax.iota(jnp.int32, LANES)
        lo = plsc.load_gather(lo_v, [bidx])
        acc = plsc.load_gather(b_v, [bidx, lo])
        for d in range(d_latent):
            fd = jnp.full((LANES,), d, jnp.int32)
            ud = plsc.load_gather(u_v, [bidx, fd])
            vd = plsc.load_gather(i_v, [bidx, fd])
            acc = acc + jnp.maximum(ud * vd, 0.0) * hs[d]
        plsc.store_scatter(o_v, [bidx, zeros], acc)
        return carry

    lax.fori_loop(0, b_per_w // LANES, group, 0)

    pltpu.sync_copy(o_v, out_hbm.at[pl.ds(wid * b_per_w, b_per_w)])


def kernel(user_indices, item_indices, embedding_user, embedding_item,
           bias_item, h):
    batch = user_indices.shape[0]
    d_latent = embedding_user.shape[1]
    num_items = bias_item.shape[0]
    assert batch % (NW * IDX_CHUNK) == 0 and num_items % LANES == 0
    b_per_w = batch // NW
    n_chunks = b_per_w // IDX_CHUNK

    ui2 = user_indices.reshape(NW * n_chunks, IDX_CHUNK)
    ii2 = item_indices.reshape(NW * n_chunks, IDX_CHUNK)
    b16 = bias_item.reshape(num_items // LANES, LANES)
    hrep = jnp.repeat(h.reshape(d_latent), LANES)

    mesh = plsc.VectorSubcoreMesh(core_axis_name="c", subcore_axis_name="s",
                                  num_cores=NUM_CORES,
                                  num_subcores=NUM_SUBCORES)
    run = pl.kernel(
        functools.partial(_fm_kernel, d_latent, b_per_w),
        out_type=jax.ShapeDtypeStruct((batch, 1), jnp.float32),
        mesh=mesh,
        compiler_params=pltpu.CompilerParams(needs_layout_passes=False,
                                             use_tc_tiling_on_sc=False),
        scratch_types=[
            pltpu.VMEM((n_chunks, IDX_CHUNK), jnp.int32),
            pltpu.VMEM((n_chunks, IDX_CHUNK), jnp.int32),
            pltpu.VMEM((n_chunks, IDX_CHUNK), jnp.int32),
            pltpu.VMEM((b_per_w,), jnp.int32),
            pltpu.VMEM((b_per_w, d_latent), jnp.float32),
            pltpu.VMEM((b_per_w, d_latent), jnp.float32),
            pltpu.VMEM((b_per_w, LANES), jnp.float32),
            pltpu.VMEM((d_latent * LANES,), jnp.float32),
            pltpu.VMEM((b_per_w, 1), jnp.float32),
            pltpu.SemaphoreType.DMA,
        ],
    )
    return run(ui2, ii2, embedding_user, embedding_item, b16, hrep)


# speedup vs baseline: 3.1531x; 3.1531x over previous
import jax
import jax.numpy as jnp
from jax import lax
from jax.experimental import pallas as pl
from jax.experimental.pallas import tpu as pltpu
from jax.experimental.pallas import tpu_sc as plsc

NUM_CORES = 2
NUM_SUBCORES = 16
LANES = 16
NW = NUM_CORES * NUM_SUBCORES
IDX_CHUNK = 128
RING = 4

_MESH = dict(core_axis_name="c", subcore_axis_name="s",
             num_cores=NUM_CORES, num_subcores=NUM_SUBCORES)


def _bias_kernel(b_per_w, ii_hbm, b16_hbm, out_hbm, ii_v, hi_v, lo_v, b_v,
                 o_v, sem):
    wid = lax.axis_index("s") * NUM_CORES + lax.axis_index("c")
    n_chunks = b_per_w // IDX_CHUNK
    grp_per_chunk = IDX_CHUNK // LANES

    pltpu.sync_copy(ii_hbm.at[pl.ds(wid * n_chunks, n_chunks)], ii_v)
    for j in range(n_chunks):
        for t in range(grp_per_chunk):
            sl = pl.ds(t * LANES, LANES)
            r = ii_v[j, sl]
            hi_v[j, sl] = lax.shift_right_logical(r, 4)
            lo_v[pl.ds((j * grp_per_chunk + t) * LANES, LANES)] = (
                jnp.bitwise_and(r, 15))
    copies = [pltpu.async_copy(b16_hbm.at[hi_v.at[j]],
                               b_v.at[pl.ds(j * IDX_CHUNK, IDX_CHUNK)], sem)
              for j in range(n_chunks)]
    for c in copies:
        c.wait()

    def group(g, carry):
        bidx = g * LANES + lax.iota(jnp.int32, LANES)
        lo = plsc.load_gather(lo_v, [bidx])
        plsc.store_scatter(o_v, [bidx], plsc.load_gather(b_v, [bidx, lo]))
        return carry

    lax.fori_loop(0, b_per_w // LANES, group, 0)
    pltpu.sync_copy(o_v, out_hbm.at[pl.ds(wid * b_per_w, b_per_w)])


def _main_kernel(d_latent, b_per_w, ui_hbm, ii_hbm, eut_hbm, eit_hbm, bg_hbm,
                 h_hbm, out_hbm, ui_v, ii_v, bg_v, h_v, ub, ib, o_v,
                 sem0, sem1, sem2, sem3):
    wid = lax.axis_index("s") * NUM_CORES + lax.axis_index("c")
    base = wid * b_per_w
    sems = [sem0, sem1, sem2, sem3]

    pltpu.sync_copy(ui_hbm.at[pl.ds(base, b_per_w)], ui_v)
    pltpu.sync_copy(ii_hbm.at[pl.ds(base, b_per_w)], ii_v)
    pltpu.sync_copy(bg_hbm.at[pl.ds(base, b_per_w)], bg_v)
    pltpu.sync_copy(h_hbm, h_v)

    h0 = h_v[pl.ds(0, LANES)]
    h1 = h_v[pl.ds(LANES, LANES)]
    iota = lax.iota(jnp.int32, LANES)
    lane0 = iota == 0

    def scal(ref, k):
        return plsc.load_gather(ref, [jnp.full((LANES,), k, jnp.int32)])[0]

    def fire(k, slot):
        cu = lax.mul(lax.div(scal(ui_v, k), 128), 128)
        ci = lax.mul(lax.div(scal(ii_v, k), 128), 128)
        pltpu.async_copy(eut_hbm.at[:, pl.ds(cu, 128)], ub.at[slot],
                         sems[slot])
        pltpu.async_copy(eit_hbm.at[:, pl.ds(ci, 128)], ib.at[slot],
                         sems[slot])

    for j in range(RING):
        fire(j, j)

    def quad(q, carry):
        for j in range(RING):
            k = q * RING + j
            pltpu.make_async_copy(eut_hbm.at[:, pl.ds(0, 128)], ub.at[j],
                                  sems[j]).wait()
            pltpu.make_async_copy(eit_hbm.at[:, pl.ds(0, 128)], ib.at[j],
                                  sems[j]).wait()
            fj = jnp.full((LANES,), j, jnp.int32)
            lu = jnp.bitwise_and(scal(ui_v, k), 127)
            li = jnp.bitwise_and(scal(ii_v, k), 127)
            flu = jnp.full((LANES,), lu, jnp.int32)
            fli = jnp.full((LANES,), li, jnp.int32)
            u0 = plsc.load_gather(ub, [fj, iota, flu])
            u1 = plsc.load_gather(ub, [fj, iota + LANES, flu])
            v0 = plsc.load_gather(ib, [fj, iota, fli])
            v1 = plsc.load_gather(ib, [fj, iota + LANES, fli])
            t = (jnp.maximum(u0 * v0, 0.0) * h0
                 + jnp.maximum(u1 * v1, 0.0) * h1)
            s = lax.reduce_sum_p.bind(t, axes=(0,)) + scal(bg_v, k)
            plsc.store_scatter(o_v, [jnp.full((LANES,), k, jnp.int32)],
                               jnp.full((LANES,), s, jnp.float32),
                               mask=lane0)

            @pl.when(k + RING < b_per_w)
            def _():
                fire(k + RING, j)
        return carry

    lax.fori_loop(0, b_per_w // RING, quad, 0)
    pltpu.sync_copy(o_v, out_hbm.at[pl.ds(base, b_per_w)])


def kernel(user_indices, item_indices, embedding_user, embedding_item,
           bias_item, h):
    import functools
    batch = user_indices.shape[0]
    d_latent = embedding_user.shape[1]
    num_items = bias_item.shape[0]
    assert batch % (NW * IDX_CHUNK) == 0 and num_items % LANES == 0
    assert d_latent == 2 * LANES
    b_per_w = batch // NW
    n_chunks = b_per_w // IDX_CHUNK

    ii2 = item_indices.reshape(NW * n_chunks, IDX_CHUNK)
    b16 = bias_item.reshape(num_items // LANES, LANES)
    h1d = h.reshape(d_latent)
    mesh = plsc.VectorSubcoreMesh(**_MESH)

    bias_g = pl.kernel(
        functools.partial(_bias_kernel, b_per_w),
        out_type=jax.ShapeDtypeStruct((batch,), jnp.float32),
        mesh=mesh,
        compiler_params=pltpu.CompilerParams(needs_layout_passes=False,
                                             use_tc_tiling_on_sc=False),
        scratch_types=[
            pltpu.VMEM((n_chunks, IDX_CHUNK), jnp.int32),
            pltpu.VMEM((n_chunks, IDX_CHUNK), jnp.int32),
            pltpu.VMEM((b_per_w,), jnp.int32),
            pltpu.VMEM((b_per_w, LANES), jnp.float32),
            pltpu.VMEM((b_per_w,), jnp.float32),
            pltpu.SemaphoreType.DMA,
        ],
    )(ii2, b16)

    out = pl.kernel(
        functools.partial(_main_kernel, d_latent, b_per_w),
        out_type=jax.ShapeDtypeStruct((batch,), jnp.float32),
        mesh=mesh,
        compiler_params=pltpu.CompilerParams(needs_layout_passes=False,
                                             use_tc_tiling_on_sc=True),
        scratch_types=[
            pltpu.VMEM((batch // NW,), jnp.int32),
            pltpu.VMEM((batch // NW,), jnp.int32),
            pltpu.VMEM((batch // NW,), jnp.float32),
            pltpu.VMEM((d_latent,), jnp.float32),
            pltpu.VMEM((RING, d_latent, 128), jnp.float32),
            pltpu.VMEM((RING, d_latent, 128), jnp.float32),
            pltpu.VMEM((batch // NW,), jnp.float32),
            pltpu.SemaphoreType.DMA,
            pltpu.SemaphoreType.DMA,
            pltpu.SemaphoreType.DMA,
            pltpu.SemaphoreType.DMA,
        ],
    )(user_indices, item_indices, embedding_user.T, embedding_item.T,
      bias_g, h1d)
    return out.reshape(batch, 1)


# speedup vs baseline: 3.5924x; 1.1393x over previous
import jax
import jax.numpy as jnp
from jax import lax
from jax.experimental import pallas as pl
from jax.experimental.pallas import tpu as pltpu
from jax.experimental.pallas import tpu_sc as plsc

NUM_CORES = 2
NUM_SUBCORES = 16
LANES = 16
NW = NUM_CORES * NUM_SUBCORES
IDX_CHUNK = 128
RING = 8

_MESH = dict(core_axis_name="c", subcore_axis_name="s",
             num_cores=NUM_CORES, num_subcores=NUM_SUBCORES)


def _bias_kernel(b_per_w, ii_hbm, b16_hbm, out_hbm, ii_v, hi_v, lo_v, b_v,
                 o_v, sem):
    wid = lax.axis_index("s") * NUM_CORES + lax.axis_index("c")
    n_chunks = b_per_w // IDX_CHUNK
    grp_per_chunk = IDX_CHUNK // LANES

    pltpu.sync_copy(ii_hbm.at[pl.ds(wid * n_chunks, n_chunks)], ii_v)
    for j in range(n_chunks):
        for t in range(grp_per_chunk):
            sl = pl.ds(t * LANES, LANES)
            r = ii_v[j, sl]
            hi_v[j, sl] = lax.shift_right_logical(r, 4)
            lo_v[pl.ds((j * grp_per_chunk + t) * LANES, LANES)] = (
                jnp.bitwise_and(r, 15))
    copies = [pltpu.async_copy(b16_hbm.at[hi_v.at[j]],
                               b_v.at[pl.ds(j * IDX_CHUNK, IDX_CHUNK)], sem)
              for j in range(n_chunks)]
    for c in copies:
        c.wait()

    def group(g, carry):
        bidx = g * LANES + lax.iota(jnp.int32, LANES)
        lo = plsc.load_gather(lo_v, [bidx])
        plsc.store_scatter(o_v, [bidx], plsc.load_gather(b_v, [bidx, lo]))
        return carry

    lax.fori_loop(0, b_per_w // LANES, group, 0)
    pltpu.sync_copy(o_v, out_hbm.at[pl.ds(wid * b_per_w, b_per_w)])


def _main_kernel(d_latent, b_per_w, ui_hbm, ii_hbm, eut_hbm, eit_hbm,
                 h_hbm, out_hbm, ui_v, ii_v, h_v, ub, ib, o_v, *sems):
    wid = lax.axis_index("s") * NUM_CORES + lax.axis_index("c")
    base = wid * b_per_w

    pltpu.sync_copy(ui_hbm.at[pl.ds(base, b_per_w)],
                    ui_v.at[pl.ds(0, b_per_w)])
    pltpu.sync_copy(ii_hbm.at[pl.ds(base, b_per_w)],
                    ii_v.at[pl.ds(0, b_per_w)])
    pltpu.sync_copy(h_hbm, h_v)

    h0 = h_v[pl.ds(0, LANES)]
    h1 = h_v[pl.ds(LANES, LANES)]
    iota = lax.iota(jnp.int32, LANES)
    lane0 = iota == 0

    def idx_vecs(k16):
        iu = plsc.load_gather(ui_v, [jnp.full((LANES,), k16, jnp.int32) + iota])
        iv = plsc.load_gather(ii_v, [jnp.full((LANES,), k16, jnp.int32) + iota])
        return iu, iv

    def fire(iu, iv, j, slot):
        cu = lax.mul(lax.div(iu[j], 128), 128)
        ci = lax.mul(lax.div(iv[j], 128), 128)
        pltpu.async_copy(eut_hbm.at[:, pl.ds(cu, 128)], ub.at[slot],
                         sems[slot])
        pltpu.async_copy(eit_hbm.at[:, pl.ds(ci, 128)], ib.at[slot],
                         sems[slot])

    iu0, iv0 = idx_vecs(0)
    for j in range(RING):
        fire(iu0, iv0, j, j)

    def quad(q, carry):
        iu, iv = idx_vecs(q * RING)
        inext, ivnext = idx_vecs(q * RING + RING)
        for j in range(RING):
            k = q * RING + j
            pltpu.make_async_copy(eut_hbm.at[:, pl.ds(0, 128)], ub.at[j],
                                  sems[j]).wait()
            pltpu.make_async_copy(eit_hbm.at[:, pl.ds(0, 128)], ib.at[j],
                                  sems[j]).wait()
            fj = jnp.full((LANES,), j, jnp.int32)
            flu = jnp.full((LANES,), jnp.bitwise_and(iu[j], 127), jnp.int32)
            fli = jnp.full((LANES,), jnp.bitwise_and(iv[j], 127), jnp.int32)
            u0 = plsc.load_gather(ub, [fj, iota, flu])
            u1 = plsc.load_gather(ub, [fj, iota + LANES, flu])
            v0 = plsc.load_gather(ib, [fj, iota, fli])
            v1 = plsc.load_gather(ib, [fj, iota + LANES, fli])
            t = (jnp.maximum(u0 * v0, 0.0) * h0
                 + jnp.maximum(u1 * v1, 0.0) * h1)
            s = lax.reduce_sum_p.bind(t, axes=(0,))
            plsc.store_scatter(o_v, [jnp.full((LANES,), k, jnp.int32)],
                               jnp.full((LANES,), s, jnp.float32),
                               mask=lane0)

            @pl.when(k + RING < b_per_w)
            def _():
                fire(inext, ivnext, j, j)
        return carry

    lax.fori_loop(0, b_per_w // RING, quad, 0)
    pltpu.sync_copy(o_v, out_hbm.at[pl.ds(base, b_per_w)])


def kernel(user_indices, item_indices, embedding_user, embedding_item,
           bias_item, h):
    import functools
    batch = user_indices.shape[0]
    d_latent = embedding_user.shape[1]
    num_items = bias_item.shape[0]
    assert batch % (NW * IDX_CHUNK) == 0 and num_items % LANES == 0
    assert d_latent == 2 * LANES
    b_per_w = batch // NW
    n_chunks = b_per_w // IDX_CHUNK

    ii2 = item_indices.reshape(NW * n_chunks, IDX_CHUNK)
    b16 = bias_item.reshape(num_items // LANES, LANES)
    h1d = h.reshape(d_latent)
    mesh = plsc.VectorSubcoreMesh(**_MESH)

    bias_g = pl.kernel(
        functools.partial(_bias_kernel, b_per_w),
        out_type=jax.ShapeDtypeStruct((batch,), jnp.float32),
        mesh=mesh,
        compiler_params=pltpu.CompilerParams(needs_layout_passes=False,
                                             use_tc_tiling_on_sc=False),
        scratch_types=[
            pltpu.VMEM((n_chunks, IDX_CHUNK), jnp.int32),
            pltpu.VMEM((n_chunks, IDX_CHUNK), jnp.int32),
            pltpu.VMEM((b_per_w,), jnp.int32),
            pltpu.VMEM((b_per_w, LANES), jnp.float32),
            pltpu.VMEM((b_per_w,), jnp.float32),
            pltpu.SemaphoreType.DMA,
        ],
    )(ii2, b16)

    partial_out = pl.kernel(
        functools.partial(_main_kernel, d_latent, b_per_w),
        out_type=jax.ShapeDtypeStruct((batch,), jnp.float32),
        mesh=mesh,
        compiler_params=pltpu.CompilerParams(needs_layout_passes=False,
                                             use_tc_tiling_on_sc=True),
        scratch_types=[
            pltpu.VMEM((b_per_w + 2 * LANES,), jnp.int32),
            pltpu.VMEM((b_per_w + 2 * LANES,), jnp.int32),
            pltpu.VMEM((d_latent,), jnp.float32),
            pltpu.VMEM((RING, d_latent, 128), jnp.float32),
            pltpu.VMEM((RING, d_latent, 128), jnp.float32),
            pltpu.VMEM((b_per_w,), jnp.float32),
        ] + [pltpu.SemaphoreType.DMA] * RING,
    )(user_indices, item_indices, embedding_user.T, embedding_item.T, h1d)
    return (partial_out + bias_g).reshape(batch, 1)


# speedup vs baseline: 3.7018x; 1.0304x over previous
import functools

import jax
import jax.numpy as jnp
from jax import lax
from jax.experimental import pallas as pl
from jax.experimental.pallas import tpu as pltpu
from jax.experimental.pallas import tpu_sc as plsc

NUM_CORES = 2
NUM_SUBCORES = 16
LANES = 16
NW = NUM_CORES * NUM_SUBCORES
RING = 8

_MESH = dict(core_axis_name="c", subcore_axis_name="s",
             num_cores=NUM_CORES, num_subcores=NUM_SUBCORES)


def _fm_kernel(d_latent, b_per_w, ui_hbm, ii_hbm, eut_hbm, eit_hbm, b1_hbm,
               h_hbm, out_hbm, ui_v, ii_v, h_v, ub, ib, bb, o_v, *sems):
    wid = lax.axis_index("s") * NUM_CORES + lax.axis_index("c")
    base = wid * b_per_w

    pltpu.sync_copy(ui_hbm.at[pl.ds(base, b_per_w)],
                    ui_v.at[pl.ds(0, b_per_w)])
    pltpu.sync_copy(ii_hbm.at[pl.ds(base, b_per_w)],
                    ii_v.at[pl.ds(0, b_per_w)])
    pltpu.sync_copy(h_hbm, h_v)

    h0 = h_v[pl.ds(0, LANES)]
    h1 = h_v[pl.ds(LANES, LANES)]
    iota = lax.iota(jnp.int32, LANES)
    lane0 = iota == 0

    def idx_vecs(k16):
        fk = jnp.full((LANES,), k16, jnp.int32) + iota
        return plsc.load_gather(ui_v, [fk]), plsc.load_gather(ii_v, [fk])

    def fire(iu, iv, j, slot):
        cu = lax.mul(lax.div(iu[j], 128), 128)
        ci = lax.mul(lax.div(iv[j], 128), 128)
        cb = lax.mul(lax.div(iv[j], LANES), LANES)
        pltpu.async_copy(eut_hbm.at[:, pl.ds(cu, 128)], ub.at[slot],
                         sems[slot])
        pltpu.async_copy(eit_hbm.at[:, pl.ds(ci, 128)], ib.at[slot],
                         sems[slot])
        pltpu.async_copy(b1_hbm.at[pl.ds(cb, LANES)], bb.at[slot],
                         sems[slot])

    iu0, iv0 = idx_vecs(0)
    for j in range(RING):
        fire(iu0, iv0, j, j)

    def step(q, carry):
        iu, iv = idx_vecs(q * RING)
        inext, ivnext = idx_vecs(q * RING + RING)
        for j in range(RING):
            k = q * RING + j
            pltpu.make_async_copy(eut_hbm.at[:, pl.ds(0, 128)], ub.at[j],
                                  sems[j]).wait()
            pltpu.make_async_copy(eit_hbm.at[:, pl.ds(0, 128)], ib.at[j],
                                  sems[j]).wait()
            pltpu.make_async_copy(b1_hbm.at[pl.ds(0, LANES)], bb.at[j],
                                  sems[j]).wait()
            fj = jnp.full((LANES,), j, jnp.int32)
            flu = jnp.full((LANES,), jnp.bitwise_and(iu[j], 127), jnp.int32)
            fli = jnp.full((LANES,), jnp.bitwise_and(iv[j], 127), jnp.int32)
            flb = jnp.full((LANES,), jnp.bitwise_and(iv[j], LANES - 1),
                           jnp.int32)
            u0 = plsc.load_gather(ub, [fj, iota, flu])
            u1 = plsc.load_gather(ub, [fj, iota + LANES, flu])
            v0 = plsc.load_gather(ib, [fj, iota, fli])
            v1 = plsc.load_gather(ib, [fj, iota + LANES, fli])
            t = (jnp.maximum(u0 * v0, 0.0) * h0
                 + jnp.maximum(u1 * v1, 0.0) * h1)
            s = (lax.reduce_sum_p.bind(t, axes=(0,))
                 + plsc.load_gather(bb, [fj, flb])[0])
            plsc.store_scatter(o_v, [jnp.full((LANES,), k, jnp.int32)],
                               jnp.full((LANES,), s, jnp.float32),
                               mask=lane0)

            @pl.when(k + RING < b_per_w)
            def _():
                fire(inext, ivnext, j, j)
        return carry

    lax.fori_loop(0, b_per_w // RING, step, 0)
    pltpu.sync_copy(o_v, out_hbm.at[pl.ds(base, b_per_w)])


def kernel(user_indices, item_indices, embedding_user, embedding_item,
           bias_item, h):
    batch = user_indices.shape[0]
    d_latent = embedding_user.shape[1]
    num_items = bias_item.shape[0]
    assert batch % (NW * RING) == 0 and num_items % LANES == 0
    assert d_latent == 2 * LANES
    b_per_w = batch // NW

    b1 = bias_item.reshape(num_items)
    h1d = h.reshape(d_latent)
    mesh = plsc.VectorSubcoreMesh(**_MESH)

    out = pl.kernel(
        functools.partial(_fm_kernel, d_latent, b_per_w),
        out_type=jax.ShapeDtypeStruct((batch,), jnp.float32),
        mesh=mesh,
        compiler_params=pltpu.CompilerParams(needs_layout_passes=False,
                                             use_tc_tiling_on_sc=True),
        scratch_types=[
            pltpu.VMEM((b_per_w + 2 * LANES,), jnp.int32),
            pltpu.VMEM((b_per_w + 2 * LANES,), jnp.int32),
            pltpu.VMEM((d_latent,), jnp.float32),
            pltpu.VMEM((RING, d_latent, 128), jnp.float32),
            pltpu.VMEM((RING, d_latent, 128), jnp.float32),
            pltpu.VMEM((RING, LANES), jnp.float32),
            pltpu.VMEM((b_per_w,), jnp.float32),
        ] + [pltpu.SemaphoreType.DMA] * RING,
    )(user_indices, item_indices, embedding_user.T, embedding_item.T,
      b1, h1d)
    return out.reshape(batch, 1)
